# Initial kernel scaffold; baseline (speedup 1.0000x reference)
import jax
import jax.numpy as jnp
from jax.experimental import pallas as pl


def kernel(x, edge_index, W1, b1, W2, b2):
    raise NotImplementedError("write your pallas kernel here")


# speedup vs baseline: 9.0532x; 9.0532x over previous
import jax
import jax.numpy as jnp
from jax import lax
from jax.experimental import pallas as pl
from jax.experimental.pallas import tpu as pltpu
from jax.experimental.pallas import tpu_sc as plsc

NC = 2
NS = 16
NW = NC * NS
LANES = 16

N = 10000
D = 128
E = 320000
N_PAD = 10240
H = N_PAD // NC
CHUNK = 128
A_ROWS = H + CHUNK
AZ_PER_TILE = A_ROWS // NS
AC_PER_TILE = H // NS
DEG_PER_TILE = N_PAD // NS
C = -(-E // (NW * CHUNK))
E_PAD = NW * C * CHUNK
C2 = E_PAD // (NS * CHUNK)

_mesh = lambda: plsc.VectorSubcoreMesh(core_axis_name="c", subcore_axis_name="s")


def _deg_body(dst_hbm, out_hbm, dst_v, ones_v, zero_v, deg_sh):
  c = lax.axis_index("c")
  s = lax.axis_index("s")
  w = c * NS + s
  pltpu.sync_copy(dst_hbm.at[w], dst_v)

  def fill(i, carry):
    ones_v[pl.ds(i * LANES, LANES)] = jnp.ones((LANES,), jnp.float32)
    return carry
  lax.fori_loop(0, CHUNK // LANES, fill, 0, unroll=8)

  def zfill(i, carry):
    zero_v[pl.ds(i * LANES, LANES)] = jnp.zeros((LANES,), jnp.float32)
    return carry
  lax.fori_loop(0, DEG_PER_TILE // LANES, zfill, 0, unroll=8)

  pltpu.sync_copy(zero_v, deg_sh.at[pl.ds(s * DEG_PER_TILE, DEG_PER_TILE)])
  plsc.subcore_barrier()

  def count(j, carry):
    pltpu.sync_copy(ones_v, deg_sh.at[dst_v.at[j]], add=True)
    return carry
  lax.fori_loop(0, C, count, 0)

  plsc.subcore_barrier()
  pltpu.sync_copy(deg_sh.at[pl.ds(s * DEG_PER_TILE, DEG_PER_TILE)],
                  out_hbm.at[c].at[pl.ds(s * DEG_PER_TILE, DEG_PER_TILE)])


def _deg_partials(dst3):
  k = pl.kernel(
      _deg_body,
      out_type=jax.ShapeDtypeStruct((NC, N_PAD), jnp.float32),
      mesh=_mesh(),
      scratch_types=[
          pltpu.VMEM((C, CHUNK), jnp.int32),
          pltpu.VMEM((CHUNK,), jnp.float32),
          pltpu.VMEM((DEG_PER_TILE,), jnp.float32),
          pltpu.VMEM_SHARED((N_PAD,), jnp.float32),
      ],
  )
  return k(dst3)


def _agg_body(g_hbm, src_hbm, dst_hbm, acc_out, src_v, dst_v, rows, acc_sh,
              sem0, sem1):
  c = lax.axis_index("c")
  s = lax.axis_index("s")
  pltpu.sync_copy(src_hbm.at[s], src_v)
  pltpu.sync_copy(dst_hbm.at[c].at[s], dst_v)

  def zero(i, carry):
    for k in range(D // LANES):
      rows[0, i, pl.ds(k * LANES, LANES)] = jnp.zeros((LANES,), jnp.float32)
    return carry
  lax.fori_loop(0, CHUNK, zero, 0, unroll=4)
  base = s * AZ_PER_TILE
  pltpu.sync_copy(rows.at[0], acc_sh.at[pl.ds(base, CHUNK)])
  pltpu.sync_copy(rows.at[0], acc_sh.at[pl.ds(base + CHUNK, CHUNK)])
  pltpu.sync_copy(rows.at[0].at[pl.ds(0, AZ_PER_TILE - 2 * CHUNK)],
                  acc_sh.at[pl.ds(base + 2 * CHUNK, AZ_PER_TILE - 2 * CHUNK)])
  plsc.subcore_barrier()

  sems = (sem0, sem1)

  pltpu.async_copy(g_hbm.at[src_v.at[0]], rows.at[0], sem0)

  def pair(p, carry):
    for b in range(2):
      j = p * 2 + b

      @pl.when(j + 1 < C2)
      def _prefetch():
        nb = 1 - b
        pltpu.async_copy(g_hbm.at[src_v.at[j + 1]], rows.at[nb], sems[nb])

      pltpu.make_async_copy(g_hbm.at[src_v.at[j]], rows.at[b], sems[b]).wait()
      pltpu.sync_copy(rows.at[b], acc_sh.at[dst_v.at[j]], add=True)
    return carry

  lax.fori_loop(0, C2 // 2, pair, 0)

  plsc.subcore_barrier()
  pltpu.sync_copy(acc_sh.at[pl.ds(s * AC_PER_TILE, AC_PER_TILE)],
                  acc_out.at[c].at[pl.ds(s * AC_PER_TILE, AC_PER_TILE)])


def _scatter_rows(g, src2, dst_off):
  k = pl.kernel(
      _agg_body,
      out_type=jax.ShapeDtypeStruct((NC, H, D), jnp.float32),
      mesh=_mesh(),
      scratch_types=[
          pltpu.VMEM((C2, CHUNK), jnp.int32),
          pltpu.VMEM((C2, CHUNK), jnp.int32),
          pltpu.VMEM((2, CHUNK, D), jnp.float32),
          pltpu.VMEM_SHARED((A_ROWS, D), jnp.float32),
          pltpu.SemaphoreType.DMA,
          pltpu.SemaphoreType.DMA,
      ],
  )
  return k(g, src2, dst_off)


BLK = 512
GRID = N_PAD // BLK


def _dinv_col(dp):
  deg = jnp.sum(dp, axis=1, keepdims=True) + 1.0
  return lax.rsqrt(deg)


def _mm1_body(x_ref, w_ref, dp_ref, h_ref, g_ref):
  h = jnp.dot(x_ref[...], w_ref[...], preferred_element_type=jnp.float32)
  dinv = _dinv_col(dp_ref[...])
  h_ref[...] = h
  g_ref[...] = h * dinv


def _mid_body(acc_ref, h_ref, dp_ref, b_ref, w_ref, h2_ref, g2_ref):
  dinv = _dinv_col(dp_ref[...])
  z = jnp.maximum(acc_ref[...] * dinv + h_ref[...] * (dinv * dinv) + b_ref[...],
                  0.0)
  h2 = jnp.dot(z, w_ref[...], preferred_element_type=jnp.float32)
  h2_ref[...] = h2
  g2_ref[...] = h2 * dinv


def _fin_body(acc_ref, h_ref, dp_ref, b_ref, o_ref):
  dinv = _dinv_col(dp_ref[...])
  o_ref[...] = acc_ref[...] * dinv + h_ref[...] * (dinv * dinv) + b_ref[...]


def _mm1(x_pad, W1, dpT):
  return pl.pallas_call(
      _mm1_body,
      grid=(GRID,),
      in_specs=[
          pl.BlockSpec((BLK, D), lambda i: (i, 0)),
          pl.BlockSpec((D, D), lambda i: (0, 0)),
          pl.BlockSpec((BLK, NC), lambda i: (i, 0)),
      ],
      out_specs=[
          pl.BlockSpec((BLK, D), lambda i: (i, 0)),
          pl.BlockSpec((BLK, D), lambda i: (i, 0)),
      ],
      out_shape=[
          jax.ShapeDtypeStruct((N_PAD, D), jnp.float32),
          jax.ShapeDtypeStruct((N_PAD, D), jnp.float32),
      ],
  )(x_pad, W1, dpT)


def _mid(acc, h1, dpT, b1, W2):
  return pl.pallas_call(
      _mid_body,
      grid=(GRID,),
      in_specs=[
          pl.BlockSpec((BLK, D), lambda i: (i, 0)),
          pl.BlockSpec((BLK, D), lambda i: (i, 0)),
          pl.BlockSpec((BLK, NC), lambda i: (i, 0)),
          pl.BlockSpec((1, D), lambda i: (0, 0)),
          pl.BlockSpec((D, D), lambda i: (0, 0)),
      ],
      out_specs=[
          pl.BlockSpec((BLK, D), lambda i: (i, 0)),
          pl.BlockSpec((BLK, D), lambda i: (i, 0)),
      ],
      out_shape=[
          jax.ShapeDtypeStruct((N_PAD, D), jnp.float32),
          jax.ShapeDtypeStruct((N_PAD, D), jnp.float32),
      ],
  )(acc, h1, dpT, b1.reshape(1, D), W2)


def _fin(acc, h2, dpT, b2):
  return pl.pallas_call(
      _fin_body,
      grid=(GRID,),
      in_specs=[
          pl.BlockSpec((BLK, D), lambda i: (i, 0)),
          pl.BlockSpec((BLK, D), lambda i: (i, 0)),
          pl.BlockSpec((BLK, NC), lambda i: (i, 0)),
          pl.BlockSpec((1, D), lambda i: (0, 0)),
      ],
      out_specs=pl.BlockSpec((BLK, D), lambda i: (i, 0)),
      out_shape=jax.ShapeDtypeStruct((N_PAD, D), jnp.float32),
  )(acc, h2, dpT, b2.reshape(1, D))


@jax.jit
def kernel(x, edge_index, W1, b1, W2, b2):
  src = edge_index[0]
  dst = edge_index[1]
  pad = E_PAD - E
  src_p = jnp.concatenate([src, jnp.full((pad,), N, jnp.int32)])
  dst_p = jnp.concatenate([dst, jnp.full((pad,), N, jnp.int32)])
  dst3 = dst_p.reshape(NW, C, CHUNK)
  dst2 = dst_p.reshape(NS, C2, CHUNK)
  src2 = src_p.reshape(NS, C2, CHUNK)
  dst_off = jnp.stack([
      jnp.where((dst2 >= c * H) & (dst2 < (c + 1) * H), dst2 - c * H, H)
      for c in range(NC)
  ]).astype(jnp.int32)

  x_pad = jnp.zeros((N_PAD, D), jnp.float32).at[:N].set(x)

  dp = _deg_partials(dst3)
  dpT = dp.T

  h1, g1 = _mm1(x_pad, W1, dpT)
  acc1 = _scatter_rows(g1, src2, dst_off).reshape(N_PAD, D)
  h2, g2 = _mid(acc1, h1, dpT, b1, W2)
  acc2 = _scatter_rows(g2, src2, dst_off).reshape(N_PAD, D)
  out = _fin(acc2, h2, dpT, b2)
  return out[:N]
